# baseline (speedup 1.0000x reference)
import functools

import jax
import jax.numpy as jnp
from jax import lax
from jax.experimental import pallas as pl
from jax.experimental.pallas import tpu as pltpu
from jax.experimental.pallas import tpu_sc as plsc

N_NODES = 100000
N_EDGES = 1600000
WIDTH = 32

NC = 2
NS = 16
NW = NC * NS

CHUNK = 128
SB = 8
BLKE = SB * CHUNK
NPW = 3136
NPAD = NW * NPW
NZS = NPAD // NS
EPW = 50176
EPAD = NW * EPW
NBLOCKS = EPW // BLKE
NBLK = NPW // 16

_mesh = lambda: plsc.VectorSubcoreMesh(core_axis_name="c", subcore_axis_name="s")


def _wid():
    return lax.axis_index("c") * NS + lax.axis_index("s")


def _zero_spmem(zbuf, accs):
    sid = lax.axis_index("s")

    def zb(i, _):
        zbuf[pl.ds(i * 16, 16)] = jnp.zeros((16,), jnp.float32)
        return 0

    lax.fori_loop(0, NZS // 16, zb, 0)
    for acc in accs:
        pltpu.sync_copy(zbuf, acc.at[pl.ds(sid * NZS, NZS)])
    plsc.subcore_barrier()


def _dump_spmem(out_hbm, kind, acc):
    cid = lax.axis_index("c")
    sid = lax.axis_index("s")
    base = (kind * NC + cid) * NPAD + sid * NZS
    pltpu.sync_copy(acc.at[pl.ds(sid * NZS, NZS)], out_hbm.at[pl.ds(base, NZS)])


def _fast_rsqrt(d):
    half = d * 0.5
    i = plsc.bitcast(d, jnp.int32)
    i = jnp.int32(0x5F3759DF) - lax.shift_right_logical(i, 1)
    y = plsc.bitcast(i, jnp.float32)
    for _ in range(3):
        y = y * (1.5 - half * y * y)
    return y


@functools.partial(
    pl.kernel,
    out_type=jax.ShapeDtypeStruct((2 * NC * NPAD,), jnp.float32),
    mesh=_mesh(),
    scratch_types=[
        [pltpu.VMEM((CHUNK,), jnp.int32)] * SB,
        [pltpu.VMEM((CHUNK,), jnp.int32)] * SB,
        pltpu.VMEM((CHUNK,), jnp.float32),
        pltpu.VMEM((NZS,), jnp.float32),
        pltpu.VMEM_SHARED((NPAD,), jnp.float32),
        pltpu.VMEM_SHARED((NPAD,), jnp.float32),
        pltpu.SemaphoreType.DMA,
        pltpu.SemaphoreType.DMA,
    ],
)
def _k_degrees(src_hbm, dst_hbm, out_hbm, sidx, didx, ones_v, zbuf, acc_o, acc_i,
               sem_i, sem_s):
    def fill(i, _):
        ones_v[pl.ds(i * 16, 16)] = jnp.ones((16,), jnp.float32)
        return 0

    lax.fori_loop(0, CHUNK // 16, fill, 0)
    _zero_spmem(zbuf, (acc_o, acc_i))

    eb = _wid() * EPW

    def body(t, _):
        o = eb + t * BLKE
        ld = []
        for j in range(SB):
            ld.append(pltpu.async_copy(
                src_hbm.at[pl.ds(o + j * CHUNK, CHUNK)], sidx[j], sem_i))
            ld.append(pltpu.async_copy(
                dst_hbm.at[pl.ds(o + j * CHUNK, CHUNK)], didx[j], sem_i))
        for d in ld:
            d.wait()
        sc = []
        for j in range(SB):
            sc.append(pltpu.async_copy(ones_v, acc_o.at[sidx[j]], sem_s, add=True))
            sc.append(pltpu.async_copy(ones_v, acc_i.at[didx[j]], sem_s, add=True))
        for d in sc:
            d.wait()
        return 0

    lax.fori_loop(0, NBLOCKS, body, 0)
    plsc.subcore_barrier()
    _dump_spmem(out_hbm, 0, acc_o)
    _dump_spmem(out_hbm, 1, acc_i)


@functools.partial(
    pl.kernel,
    out_type=(
        jax.ShapeDtypeStruct((NPAD,), jnp.float32),
        jax.ShapeDtypeStruct((NPAD,), jnp.float32),
        jax.ShapeDtypeStruct((NPAD,), jnp.float32),
    ),
    mesh=_mesh(),
    compiler_params=pltpu.CompilerParams(needs_layout_passes=False),
    scratch_types=[
        pltpu.VMEM((NPW,), jnp.float32),
        pltpu.VMEM((NPW,), jnp.float32),
        pltpu.VMEM((NPW,), jnp.float32),
        pltpu.VMEM((NPW,), jnp.float32),
        pltpu.VMEM((NPW,), jnp.float32),
    ],
)
def _k_norms(xp_hbm, degs_hbm, p_hbm, q_hbm, r_hbm, a0, a1, xv, ob, qb):
    nb = _wid() * NPW
    sl = pl.ds(nb, NPW)

    pltpu.sync_copy(degs_hbm.at[pl.ds(nb, NPW)], a0)
    pltpu.sync_copy(degs_hbm.at[pl.ds(NPAD + nb, NPW)], a1)
    pltpu.sync_copy(xp_hbm.at[sl], xv)

    def body_r(i, _):
        blk = pl.ds(i * 16, 16)
        d = a0[blk] + a1[blk]
        y = _fast_rsqrt(jnp.where(d > 0.0, d, 1.0))
        ob[blk] = y
        qb[blk] = y * xv[blk]
        return 0

    lax.fori_loop(0, NBLK, body_r, 0)
    pltpu.sync_copy(ob, r_hbm.at[sl])
    pltpu.sync_copy(qb, p_hbm.at[sl])

    pltpu.sync_copy(degs_hbm.at[pl.ds(2 * NPAD + nb, NPW)], a0)
    pltpu.sync_copy(degs_hbm.at[pl.ds(3 * NPAD + nb, NPW)], a1)

    def body_q(i, _):
        blk = pl.ds(i * 16, 16)
        d = a0[blk] + a1[blk]
        ob[blk] = _fast_rsqrt(jnp.where(d > 0.0, d, 1.0))
        return 0

    lax.fori_loop(0, NBLK, body_q, 0)
    pltpu.sync_copy(ob, q_hbm.at[sl])


@functools.partial(
    pl.kernel,
    out_type=(
        jax.ShapeDtypeStruct((2 * NC * NPAD,), jnp.float32),
        jax.ShapeDtypeStruct((NPAD,), jnp.float32),
        jax.ShapeDtypeStruct((NPAD,), jnp.float32),
    ),
    mesh=_mesh(),
    compiler_params=pltpu.CompilerParams(needs_layout_passes=False),
    scratch_types=[
        [pltpu.VMEM((CHUNK,), jnp.int32)] * SB,
        [pltpu.VMEM((CHUNK,), jnp.int32)] * SB,
        [pltpu.VMEM((CHUNK,), jnp.float32)] * SB,
        [pltpu.VMEM((CHUNK,), jnp.float32)] * SB,
        pltpu.VMEM((NZS,), jnp.float32),
        pltpu.VMEM((NZS,), jnp.float32),
        pltpu.VMEM((NZS,), jnp.float32),
        pltpu.VMEM_SHARED((NPAD,), jnp.float32),
        pltpu.VMEM_SHARED((NPAD,), jnp.float32),
        pltpu.VMEM_SHARED((NPAD,), jnp.float32),
        pltpu.VMEM_SHARED((NPAD,), jnp.float32),
        pltpu.SemaphoreType.DMA,
        pltpu.SemaphoreType.DMA,
        pltpu.SemaphoreType.DMA,
    ],
)
def _k_edges(src_hbm, dst_hbm, xp_hbm, degs_hbm, out_hbm, q_hbm, r_hbm,
             sidx, didx, pv, qv, zbuf, na, nb_, acc_s, acc_v, p_sp, q_sp,
             sem_i, sem_g, sem_s):
    cid = lax.axis_index("c")
    sid = lax.axis_index("s")
    tb = sid * NZS
    sl16 = pl.ds(tb, NZS)

    pltpu.sync_copy(degs_hbm.at[pl.ds(tb, NZS)], na)
    pltpu.sync_copy(degs_hbm.at[pl.ds(NPAD + tb, NZS)], nb_)
    pltpu.sync_copy(xp_hbm.at[sl16], zbuf)

    def nrm_r(i, _):
        blk = pl.ds(i * 16, 16)
        d = na[blk] + nb_[blk]
        y = _fast_rsqrt(jnp.where(d > 0.0, d, 1.0))
        na[blk] = y
        zbuf[blk] = y * zbuf[blk]
        return 0

    lax.fori_loop(0, NZS // 16, nrm_r, 0)
    pltpu.sync_copy(zbuf, p_sp.at[sl16])

    @pl.when(cid == 0)
    def _():
        pltpu.sync_copy(na, r_hbm.at[sl16])

    pltpu.sync_copy(degs_hbm.at[pl.ds(2 * NPAD + tb, NZS)], na)
    pltpu.sync_copy(degs_hbm.at[pl.ds(3 * NPAD + tb, NZS)], nb_)

    def nrm_q(i, _):
        blk = pl.ds(i * 16, 16)
        d = na[blk] + nb_[blk]
        na[blk] = _fast_rsqrt(jnp.where(d > 0.0, d, 1.0))
        return 0

    lax.fori_loop(0, NZS // 16, nrm_q, 0)
    pltpu.sync_copy(na, q_sp.at[sl16])

    @pl.when(cid == 0)
    def _():
        pltpu.sync_copy(na, q_hbm.at[sl16])

    _zero_spmem(zbuf, (acc_s, acc_v))

    eb = _wid() * EPW

    def body(t, _):
        o = eb + t * BLKE
        ld = []
        for j in range(SB):
            ld.append(pltpu.async_copy(
                src_hbm.at[pl.ds(o + j * CHUNK, CHUNK)], sidx[j], sem_i))
            ld.append(pltpu.async_copy(
                dst_hbm.at[pl.ds(o + j * CHUNK, CHUNK)], didx[j], sem_i))
        for d in ld:
            d.wait()
        gs = []
        for j in range(SB):
            gs.append(pltpu.async_copy(p_sp.at[sidx[j]], pv[j], sem_g))
            gs.append(pltpu.async_copy(q_sp.at[didx[j]], qv[j], sem_g))
        for d in gs:
            d.wait()
        sc = []
        for j in range(SB):
            sc.append(pltpu.async_copy(pv[j], acc_s.at[didx[j]], sem_s, add=True))
            sc.append(pltpu.async_copy(qv[j], acc_v.at[sidx[j]], sem_s, add=True))
        for d in sc:
            d.wait()
        return 0

    lax.fori_loop(0, NBLOCKS, body, 0)
    plsc.subcore_barrier()
    _dump_spmem(out_hbm, 0, acc_s)
    _dump_spmem(out_hbm, 1, acc_v)


@functools.partial(
    pl.kernel,
    out_type=jax.ShapeDtypeStruct((NC * 16,), jnp.float32),
    mesh=_mesh(),
    scratch_types=[
        pltpu.VMEM((NPW,), jnp.float32),
        pltpu.VMEM((NPW,), jnp.float32),
        pltpu.VMEM((NPW,), jnp.float32),
        pltpu.VMEM((NPW,), jnp.float32),
        pltpu.VMEM((NPW,), jnp.float32),
        pltpu.VMEM((NPW,), jnp.float32),
        pltpu.VMEM((WIDTH,), jnp.float32),
        pltpu.VMEM((WIDTH,), jnp.float32),
        pltpu.VMEM((WIDTH,), jnp.float32),
        pltpu.VMEM((16,), jnp.float32),
        pltpu.VMEM_SHARED((NS * 16,), jnp.float32),
    ],
)
def _k_reduce(sv_hbm, q_hbm, r_hbm, w1_hbm, b1_hbm, w2_hbm, out_hbm,
              sa0, sa1, va0, va1, qv, rv, w1v, b1v, w2v, pbuf, stage):
    cid = lax.axis_index("c")
    sid = lax.axis_index("s")
    nb = _wid() * NPW
    sl = pl.ds(nb, NPW)

    pltpu.sync_copy(sv_hbm.at[pl.ds(nb, NPW)], sa0)
    pltpu.sync_copy(sv_hbm.at[pl.ds(NPAD + nb, NPW)], sa1)
    pltpu.sync_copy(sv_hbm.at[pl.ds(2 * NPAD + nb, NPW)], va0)
    pltpu.sync_copy(sv_hbm.at[pl.ds(3 * NPAD + nb, NPW)], va1)
    pltpu.sync_copy(q_hbm.at[pl.ds(nb, NPW)], qv)
    pltpu.sync_copy(r_hbm.at[pl.ds(nb, NPW)], rv)
    pltpu.sync_copy(w1_hbm, w1v)
    pltpu.sync_copy(b1_hbm, b1v)
    pltpu.sync_copy(w2_hbm, w2v)

    def _scalars(ref):
        vals = []
        for h in range(WIDTH // 16):
            blk = ref[pl.ds(h * 16, 16)]
            vals.extend(blk[j] for j in range(16))
        return vals

    w1s = _scalars(w1v)
    b1s = _scalars(b1v)
    w2s = _scalars(w2v)
    lanes = lax.iota(jnp.int32, 16)

    pbuf[...] = jnp.zeros((16,), jnp.float32)

    def body(i, _):
        blk = pl.ds(i * 16, 16)
        s = qv[blk] * (sa0[blk] + sa1[blk])
        v = rv[blk] * (va0[blk] + va1[blk])
        g = nb + i * 16 + lanes
        v = jnp.where(g < N_NODES, v, 0.0)
        t = jnp.zeros((16,), jnp.float32)
        for j in range(WIDTH):
            t = t + jnp.maximum(s * w1s[j] + b1s[j], 0.0) * w2s[j]
        pbuf[...] = pbuf[...] + v * t
        return 0

    lax.fori_loop(0, NBLK, body, 0)
    pltpu.sync_copy(pbuf, stage.at[pl.ds(sid * 16, 16)])
    plsc.subcore_barrier()

    @pl.when(sid == 0)
    def _():
        acc = jnp.zeros((16,), jnp.float32)
        for ss in range(NS):
            pltpu.sync_copy(stage.at[pl.ds(ss * 16, 16)], pbuf)
            acc = acc + pbuf[...]
        pbuf[...] = acc
        pltpu.sync_copy(pbuf, out_hbm.at[pl.ds(cid * 16, 16)])


def kernel(x, edge_index, W1, b1, W2, b2):
    src = edge_index[0].astype(jnp.int32)
    dst = edge_index[1].astype(jnp.int32)
    fill = jnp.full((EPAD - N_EDGES,), NPAD - 1, jnp.int32)
    srcp = jnp.concatenate([src, fill])
    dstp = jnp.concatenate([dst, fill])
    xp = jnp.pad(x[:, 0], (0, NPAD - N_NODES))

    degs = _k_degrees(srcp, dstp)
    sv, q, r = _k_edges(srcp, dstp, xp, degs)
    part = _k_reduce(sv, q, r, W1.reshape(WIDTH), b1, W2.reshape(WIDTH))
    return jnp.sum(part) / N_NODES + b2[0]


# speedup vs baseline: 1.0291x; 1.0291x over previous
import functools

import jax
import jax.numpy as jnp
from jax import lax
from jax.experimental import pallas as pl
from jax.experimental.pallas import tpu as pltpu
from jax.experimental.pallas import tpu_sc as plsc

N_NODES = 100000
N_EDGES = 1600000
WIDTH = 32

NC = 2
NS = 16
NW = NC * NS

CHUNK = 128
SB = 8
BLKE = SB * CHUNK
NPW = 3136
NPAD = NW * NPW
NZS = NPAD // NS
EPW = 50176
EPAD = NW * EPW
NBLOCKS = EPW // BLKE
NBLK = NPW // 16

_mesh = lambda: plsc.VectorSubcoreMesh(core_axis_name="c", subcore_axis_name="s")


def _wid():
    return lax.axis_index("c") * NS + lax.axis_index("s")


def _zero_spmem(zbuf, accs):
    sid = lax.axis_index("s")

    def zb(i, _):
        zbuf[pl.ds(i * 16, 16)] = jnp.zeros((16,), jnp.float32)
        return 0

    lax.fori_loop(0, NZS // 16, zb, 0)
    for acc in accs:
        pltpu.sync_copy(zbuf, acc.at[pl.ds(sid * NZS, NZS)])
    plsc.subcore_barrier()


def _dump_spmem(out_hbm, kind, acc):
    cid = lax.axis_index("c")
    sid = lax.axis_index("s")
    base = (kind * NC + cid) * NPAD + sid * NZS
    pltpu.sync_copy(acc.at[pl.ds(sid * NZS, NZS)], out_hbm.at[pl.ds(base, NZS)])


def _fast_rsqrt(d):
    half = d * 0.5
    i = plsc.bitcast(d, jnp.int32)
    i = jnp.int32(0x5F3759DF) - lax.shift_right_logical(i, 1)
    y = plsc.bitcast(i, jnp.float32)
    for _ in range(3):
        y = y * (1.5 - half * y * y)
    return y


@functools.partial(
    pl.kernel,
    out_type=jax.ShapeDtypeStruct((2 * NC * NPAD,), jnp.float32),
    mesh=_mesh(),
    scratch_types=[
        [pltpu.VMEM((CHUNK,), jnp.int32)] * SB,
        [pltpu.VMEM((CHUNK,), jnp.int32)] * SB,
        pltpu.VMEM((CHUNK,), jnp.float32),
        pltpu.VMEM((NZS,), jnp.float32),
        pltpu.VMEM_SHARED((NPAD,), jnp.float32),
        pltpu.VMEM_SHARED((NPAD,), jnp.float32),
        pltpu.SemaphoreType.DMA,
        pltpu.SemaphoreType.DMA,
    ],
)
def _k_degrees(src_hbm, dst_hbm, out_hbm, sidx, didx, ones_v, zbuf, acc_o, acc_i,
               sem_i, sem_s):
    def fill(i, _):
        ones_v[pl.ds(i * 16, 16)] = jnp.ones((16,), jnp.float32)
        return 0

    lax.fori_loop(0, CHUNK // 16, fill, 0)
    _zero_spmem(zbuf, (acc_o, acc_i))

    eb = _wid() * EPW

    def body(t, _):
        o = eb + t * BLKE
        ld = []
        for j in range(SB):
            ld.append(pltpu.async_copy(
                src_hbm.at[pl.ds(o + j * CHUNK, CHUNK)], sidx[j], sem_i))
            ld.append(pltpu.async_copy(
                dst_hbm.at[pl.ds(o + j * CHUNK, CHUNK)], didx[j], sem_i))
        for d in ld:
            d.wait()
        sc = []
        for j in range(SB):
            sc.append(pltpu.async_copy(ones_v, acc_o.at[sidx[j]], sem_s, add=True))
            sc.append(pltpu.async_copy(ones_v, acc_i.at[didx[j]], sem_s, add=True))
        for d in sc:
            d.wait()
        return 0

    lax.fori_loop(0, NBLOCKS, body, 0)
    plsc.subcore_barrier()
    _dump_spmem(out_hbm, 0, acc_o)
    _dump_spmem(out_hbm, 1, acc_i)


@functools.partial(
    pl.kernel,
    out_type=(
        jax.ShapeDtypeStruct((NPAD,), jnp.float32),
        jax.ShapeDtypeStruct((NPAD,), jnp.float32),
        jax.ShapeDtypeStruct((NPAD,), jnp.float32),
    ),
    mesh=_mesh(),
    compiler_params=pltpu.CompilerParams(needs_layout_passes=False),
    scratch_types=[
        pltpu.VMEM((NPW,), jnp.float32),
        pltpu.VMEM((NPW,), jnp.float32),
        pltpu.VMEM((NPW,), jnp.float32),
        pltpu.VMEM((NPW,), jnp.float32),
        pltpu.VMEM((NPW,), jnp.float32),
    ],
)
def _k_norms(xp_hbm, degs_hbm, p_hbm, q_hbm, r_hbm, a0, a1, xv, ob, qb):
    nb = _wid() * NPW
    sl = pl.ds(nb, NPW)

    pltpu.sync_copy(degs_hbm.at[pl.ds(nb, NPW)], a0)
    pltpu.sync_copy(degs_hbm.at[pl.ds(NPAD + nb, NPW)], a1)
    pltpu.sync_copy(xp_hbm.at[sl], xv)

    def body_r(i, _):
        blk = pl.ds(i * 16, 16)
        d = a0[blk] + a1[blk]
        y = _fast_rsqrt(jnp.where(d > 0.0, d, 1.0))
        ob[blk] = y
        qb[blk] = y * xv[blk]
        return 0

    lax.fori_loop(0, NBLK, body_r, 0)
    pltpu.sync_copy(ob, r_hbm.at[sl])
    pltpu.sync_copy(qb, p_hbm.at[sl])

    pltpu.sync_copy(degs_hbm.at[pl.ds(2 * NPAD + nb, NPW)], a0)
    pltpu.sync_copy(degs_hbm.at[pl.ds(3 * NPAD + nb, NPW)], a1)

    def body_q(i, _):
        blk = pl.ds(i * 16, 16)
        d = a0[blk] + a1[blk]
        ob[blk] = _fast_rsqrt(jnp.where(d > 0.0, d, 1.0))
        return 0

    lax.fori_loop(0, NBLK, body_q, 0)
    pltpu.sync_copy(ob, q_hbm.at[sl])


@functools.partial(
    pl.kernel,
    out_type=jax.ShapeDtypeStruct((2 * NC * NPAD,), jnp.float32),
    mesh=_mesh(),
    scratch_types=[
        [pltpu.VMEM((CHUNK,), jnp.int32)] * SB,
        [pltpu.VMEM((CHUNK,), jnp.int32)] * SB,
        [pltpu.VMEM((CHUNK,), jnp.float32)] * SB,
        [pltpu.VMEM((CHUNK,), jnp.float32)] * SB,
        pltpu.VMEM((NZS,), jnp.float32),
        pltpu.VMEM_SHARED((NPAD,), jnp.float32),
        pltpu.VMEM_SHARED((NPAD,), jnp.float32),
        pltpu.VMEM_SHARED((NPAD,), jnp.float32),
        pltpu.VMEM_SHARED((NPAD,), jnp.float32),
        pltpu.SemaphoreType.DMA,
        pltpu.SemaphoreType.DMA,
        pltpu.SemaphoreType.DMA,
    ],
)
def _k_edges(src_hbm, dst_hbm, p_hbm, q_hbm, out_hbm,
             sidx, didx, pv, qv, zbuf, acc_s, acc_v, p_sp, q_sp, sem_i, sem_g, sem_s):
    sid = lax.axis_index("s")
    sl16 = pl.ds(sid * NZS, NZS)
    pltpu.sync_copy(p_hbm.at[sl16], zbuf)
    pltpu.sync_copy(zbuf, p_sp.at[sl16])
    pltpu.sync_copy(q_hbm.at[sl16], zbuf)
    pltpu.sync_copy(zbuf, q_sp.at[sl16])
    _zero_spmem(zbuf, (acc_s, acc_v))

    eb = _wid() * EPW

    def body(t, _):
        o = eb + t * BLKE
        ld = []
        for j in range(SB):
            ld.append(pltpu.async_copy(
                src_hbm.at[pl.ds(o + j * CHUNK, CHUNK)], sidx[j], sem_i))
            ld.append(pltpu.async_copy(
                dst_hbm.at[pl.ds(o + j * CHUNK, CHUNK)], didx[j], sem_i))
        for d in ld:
            d.wait()
        gp = [pltpu.async_copy(p_sp.at[sidx[j]], pv[j], sem_g) for j in range(SB)]
        for d in gp:
            d.wait()
        sp = [pltpu.async_copy(pv[j], acc_s.at[didx[j]], sem_s, add=True)
              for j in range(SB)]
        gq = [pltpu.async_copy(q_sp.at[didx[j]], qv[j], sem_g) for j in range(SB)]
        for d in gq:
            d.wait()
        sq = [pltpu.async_copy(qv[j], acc_v.at[sidx[j]], sem_s, add=True)
              for j in range(SB)]
        for d in sp + sq:
            d.wait()
        return 0

    lax.fori_loop(0, NBLOCKS, body, 0)
    plsc.subcore_barrier()
    _dump_spmem(out_hbm, 0, acc_s)
    _dump_spmem(out_hbm, 1, acc_v)


@functools.partial(
    pl.kernel,
    out_type=jax.ShapeDtypeStruct((NC * 16,), jnp.float32),
    mesh=_mesh(),
    scratch_types=[
        pltpu.VMEM((NPW,), jnp.float32),
        pltpu.VMEM((NPW,), jnp.float32),
        pltpu.VMEM((NPW,), jnp.float32),
        pltpu.VMEM((NPW,), jnp.float32),
        pltpu.VMEM((NPW,), jnp.float32),
        pltpu.VMEM((NPW,), jnp.float32),
        pltpu.VMEM((WIDTH,), jnp.float32),
        pltpu.VMEM((WIDTH,), jnp.float32),
        pltpu.VMEM((WIDTH,), jnp.float32),
        pltpu.VMEM((16,), jnp.float32),
        pltpu.VMEM_SHARED((NS * 16,), jnp.float32),
    ],
)
def _k_reduce(sv_hbm, q_hbm, r_hbm, w1_hbm, b1_hbm, w2_hbm, out_hbm,
              sa0, sa1, va0, va1, qv, rv, w1v, b1v, w2v, pbuf, stage):
    cid = lax.axis_index("c")
    sid = lax.axis_index("s")
    nb = _wid() * NPW
    sl = pl.ds(nb, NPW)

    pltpu.sync_copy(sv_hbm.at[pl.ds(nb, NPW)], sa0)
    pltpu.sync_copy(sv_hbm.at[pl.ds(NPAD + nb, NPW)], sa1)
    pltpu.sync_copy(sv_hbm.at[pl.ds(2 * NPAD + nb, NPW)], va0)
    pltpu.sync_copy(sv_hbm.at[pl.ds(3 * NPAD + nb, NPW)], va1)
    pltpu.sync_copy(q_hbm.at[pl.ds(nb, NPW)], qv)
    pltpu.sync_copy(r_hbm.at[pl.ds(nb, NPW)], rv)
    pltpu.sync_copy(w1_hbm, w1v)
    pltpu.sync_copy(b1_hbm, b1v)
    pltpu.sync_copy(w2_hbm, w2v)

    def _scalars(ref):
        vals = []
        for h in range(WIDTH // 16):
            blk = ref[pl.ds(h * 16, 16)]
            vals.extend(blk[j] for j in range(16))
        return vals

    w1s = _scalars(w1v)
    b1s = _scalars(b1v)
    w2s = _scalars(w2v)
    lanes = lax.iota(jnp.int32, 16)

    pbuf[...] = jnp.zeros((16,), jnp.float32)

    def body(i, _):
        blk = pl.ds(i * 16, 16)
        s = qv[blk] * (sa0[blk] + sa1[blk])
        v = rv[blk] * (va0[blk] + va1[blk])
        g = nb + i * 16 + lanes
        v = jnp.where(g < N_NODES, v, 0.0)
        t = jnp.zeros((16,), jnp.float32)
        for j in range(WIDTH):
            t = t + jnp.maximum(s * w1s[j] + b1s[j], 0.0) * w2s[j]
        pbuf[...] = pbuf[...] + v * t
        return 0

    lax.fori_loop(0, NBLK, body, 0)
    pltpu.sync_copy(pbuf, stage.at[pl.ds(sid * 16, 16)])
    plsc.subcore_barrier()

    @pl.when(sid == 0)
    def _():
        acc = jnp.zeros((16,), jnp.float32)
        for ss in range(NS):
            pltpu.sync_copy(stage.at[pl.ds(ss * 16, 16)], pbuf)
            acc = acc + pbuf[...]
        pbuf[...] = acc
        pltpu.sync_copy(pbuf, out_hbm.at[pl.ds(cid * 16, 16)])


def kernel(x, edge_index, W1, b1, W2, b2):
    src = edge_index[0].astype(jnp.int32)
    dst = edge_index[1].astype(jnp.int32)
    fill = jnp.full((EPAD - N_EDGES,), NPAD - 1, jnp.int32)
    srcp = jnp.concatenate([src, fill])
    dstp = jnp.concatenate([dst, fill])
    xp = jnp.pad(x[:, 0], (0, NPAD - N_NODES))

    degs = _k_degrees(srcp, dstp)
    p, q, r = _k_norms(xp, degs)
    sv = _k_edges(srcp, dstp, p, q)
    part = _k_reduce(sv, q, r, W1.reshape(WIDTH), b1, W2.reshape(WIDTH))
    return jnp.sum(part) / N_NODES + b2[0]


# speedup vs baseline: 1.1554x; 1.1226x over previous
import functools

import jax
import jax.numpy as jnp
from jax import lax
from jax.experimental import pallas as pl
from jax.experimental.pallas import tpu as pltpu
from jax.experimental.pallas import tpu_sc as plsc

N_NODES = 100000
N_EDGES = 1600000
WIDTH = 32

NC = 2
NS = 16
NW = NC * NS

CHUNK = 128
SB = 8
BLKE = SB * CHUNK
NPW = 3136
NPAD = NW * NPW
NZS = NPAD // NS
EPW = 50176
EPAD = NW * EPW
NBLOCKS = EPW // BLKE
NBLK = NPW // 16

_mesh = lambda: plsc.VectorSubcoreMesh(core_axis_name="c", subcore_axis_name="s")


def _wid():
    return lax.axis_index("c") * NS + lax.axis_index("s")


def _zero_spmem(zbuf, accs):
    sid = lax.axis_index("s")

    def zb(i, _):
        zbuf[pl.ds(i * 16, 16)] = jnp.zeros((16,), jnp.float32)
        return 0

    lax.fori_loop(0, NZS // 16, zb, 0)
    for acc in accs:
        pltpu.sync_copy(zbuf, acc.at[pl.ds(sid * NZS, NZS)])
    plsc.subcore_barrier()


def _dump_spmem(out_hbm, kind, acc):
    cid = lax.axis_index("c")
    sid = lax.axis_index("s")
    base = (kind * NC + cid) * NPAD + sid * NZS
    pltpu.sync_copy(acc.at[pl.ds(sid * NZS, NZS)], out_hbm.at[pl.ds(base, NZS)])


def _fast_rsqrt(d):
    half = d * 0.5
    i = plsc.bitcast(d, jnp.int32)
    i = jnp.int32(0x5F3759DF) - lax.shift_right_logical(i, 1)
    y = plsc.bitcast(i, jnp.float32)
    for _ in range(3):
        y = y * (1.5 - half * y * y)
    return y


@functools.partial(
    pl.kernel,
    out_type=jax.ShapeDtypeStruct((2 * NC * NPAD,), jnp.float32),
    mesh=_mesh(),
    scratch_types=[
        [pltpu.VMEM((CHUNK,), jnp.int32)] * SB,
        [pltpu.VMEM((CHUNK,), jnp.int32)] * SB,
        pltpu.VMEM((CHUNK,), jnp.float32),
        pltpu.VMEM((NZS,), jnp.float32),
        pltpu.VMEM_SHARED((NPAD,), jnp.float32),
        pltpu.VMEM_SHARED((NPAD,), jnp.float32),
        pltpu.SemaphoreType.DMA,
        pltpu.SemaphoreType.DMA,
    ],
)
def _k_degrees(src_hbm, dst_hbm, out_hbm, sidx, didx, ones_v, zbuf, acc_o, acc_i,
               sem_i, sem_s):
    def fill(i, _):
        ones_v[pl.ds(i * 16, 16)] = jnp.ones((16,), jnp.float32)
        return 0

    lax.fori_loop(0, CHUNK // 16, fill, 0)
    _zero_spmem(zbuf, (acc_o, acc_i))

    eb = _wid() * EPW

    def body(t, _):
        o = eb + t * BLKE
        ld = []
        for j in range(SB):
            ld.append(pltpu.async_copy(
                src_hbm.at[pl.ds(o + j * CHUNK, CHUNK)], sidx[j], sem_i))
            ld.append(pltpu.async_copy(
                dst_hbm.at[pl.ds(o + j * CHUNK, CHUNK)], didx[j], sem_i))
        for d in ld:
            d.wait()
        sc = []
        for j in range(SB):
            sc.append(pltpu.async_copy(ones_v, acc_o.at[sidx[j]], sem_s, add=True))
            sc.append(pltpu.async_copy(ones_v, acc_i.at[didx[j]], sem_s, add=True))
        for d in sc:
            d.wait()
        return 0

    lax.fori_loop(0, NBLOCKS, body, 0)
    plsc.subcore_barrier()
    _dump_spmem(out_hbm, 0, acc_o)
    _dump_spmem(out_hbm, 1, acc_i)


NROW = NPAD // 128


def _refined_rsqrt(d):
    y = jax.lax.rsqrt(d)
    return y * (1.5 - 0.5 * d * y * y)


def _k_norms_body(xp_ref, degs_ref, p_ref, q_ref, r_ref):
    do = degs_ref[0] + degs_ref[1]
    di = degs_ref[2] + degs_ref[3]
    r = _refined_rsqrt(jnp.where(do > 0.0, do, 1.0))
    q_ref[...] = _refined_rsqrt(jnp.where(di > 0.0, di, 1.0))
    r_ref[...] = r
    p_ref[...] = r * xp_ref[...]


def _k_norms(xp, degs):
    return pl.pallas_call(
        _k_norms_body,
        out_shape=(
            jax.ShapeDtypeStruct((NROW, 128), jnp.float32),
            jax.ShapeDtypeStruct((NROW, 128), jnp.float32),
            jax.ShapeDtypeStruct((NROW, 128), jnp.float32),
        ),
    )(xp.reshape(NROW, 128), degs.reshape(4, NROW, 128))


@functools.partial(
    pl.kernel,
    out_type=jax.ShapeDtypeStruct((2 * NC * NPAD,), jnp.float32),
    mesh=_mesh(),
    scratch_types=[
        [pltpu.VMEM((CHUNK,), jnp.int32)] * SB,
        [pltpu.VMEM((CHUNK,), jnp.int32)] * SB,
        [pltpu.VMEM((CHUNK,), jnp.float32)] * SB,
        [pltpu.VMEM((CHUNK,), jnp.float32)] * SB,
        pltpu.VMEM((NZS,), jnp.float32),
        pltpu.VMEM_SHARED((NPAD,), jnp.float32),
        pltpu.VMEM_SHARED((NPAD,), jnp.float32),
        pltpu.VMEM_SHARED((NPAD,), jnp.float32),
        pltpu.VMEM_SHARED((NPAD,), jnp.float32),
        pltpu.SemaphoreType.DMA,
        pltpu.SemaphoreType.DMA,
        pltpu.SemaphoreType.DMA,
    ],
)
def _k_edges(src_hbm, dst_hbm, p_hbm, q_hbm, out_hbm,
             sidx, didx, pv, qv, zbuf, acc_s, acc_v, p_sp, q_sp, sem_i, sem_g, sem_s):
    sid = lax.axis_index("s")
    sl16 = pl.ds(sid * NZS, NZS)
    pltpu.sync_copy(p_hbm.at[sl16], zbuf)
    pltpu.sync_copy(zbuf, p_sp.at[sl16])
    pltpu.sync_copy(q_hbm.at[sl16], zbuf)
    pltpu.sync_copy(zbuf, q_sp.at[sl16])
    _zero_spmem(zbuf, (acc_s, acc_v))

    eb = _wid() * EPW

    def body(t, _):
        o = eb + t * BLKE
        ld = []
        for j in range(SB):
            ld.append(pltpu.async_copy(
                src_hbm.at[pl.ds(o + j * CHUNK, CHUNK)], sidx[j], sem_i))
            ld.append(pltpu.async_copy(
                dst_hbm.at[pl.ds(o + j * CHUNK, CHUNK)], didx[j], sem_i))
        for d in ld:
            d.wait()
        gp = [pltpu.async_copy(p_sp.at[sidx[j]], pv[j], sem_g) for j in range(SB)]
        for d in gp:
            d.wait()
        sp = [pltpu.async_copy(pv[j], acc_s.at[didx[j]], sem_s, add=True)
              for j in range(SB)]
        gq = [pltpu.async_copy(q_sp.at[didx[j]], qv[j], sem_g) for j in range(SB)]
        for d in gq:
            d.wait()
        sq = [pltpu.async_copy(qv[j], acc_v.at[sidx[j]], sem_s, add=True)
              for j in range(SB)]
        for d in sp + sq:
            d.wait()
        return 0

    lax.fori_loop(0, NBLOCKS, body, 0)
    plsc.subcore_barrier()
    _dump_spmem(out_hbm, 0, acc_s)
    _dump_spmem(out_hbm, 1, acc_v)


def _k_reduce_body(sv_ref, q_ref, r_ref, w1_ref, b1_ref, w2_ref, b2_ref, out_ref):
    s = q_ref[...] * (sv_ref[0] + sv_ref[1])
    v = r_ref[...] * (sv_ref[2] + sv_ref[3])
    g = (jax.lax.broadcasted_iota(jnp.int32, (NROW, 128), 0) * 128
         + jax.lax.broadcasted_iota(jnp.int32, (NROW, 128), 1))
    v = jnp.where(g < N_NODES, v, 0.0)
    t = jnp.zeros((NROW, 128), jnp.float32)
    for j in range(WIDTH):
        t = t + jnp.maximum(s * w1_ref[j] + b1_ref[j], 0.0) * w2_ref[j]
    out_ref[0, 0] = jnp.sum(v * t) / N_NODES + b2_ref[0]


def _k_reduce(sv, q, r, w1, b1, w2, b2):
    return pl.pallas_call(
        _k_reduce_body,
        in_specs=[
            pl.BlockSpec(memory_space=pltpu.MemorySpace.VMEM),
            pl.BlockSpec(memory_space=pltpu.MemorySpace.VMEM),
            pl.BlockSpec(memory_space=pltpu.MemorySpace.VMEM),
            pl.BlockSpec(memory_space=pltpu.MemorySpace.SMEM),
            pl.BlockSpec(memory_space=pltpu.MemorySpace.SMEM),
            pl.BlockSpec(memory_space=pltpu.MemorySpace.SMEM),
            pl.BlockSpec(memory_space=pltpu.MemorySpace.SMEM),
        ],
        out_specs=pl.BlockSpec(memory_space=pltpu.MemorySpace.SMEM),
        out_shape=jax.ShapeDtypeStruct((1, 1), jnp.float32),
    )(sv.reshape(4, NROW, 128), q, r, w1, b1, w2, b2)


def kernel(x, edge_index, W1, b1, W2, b2):
    src = edge_index[0].astype(jnp.int32)
    dst = edge_index[1].astype(jnp.int32)
    fill = jnp.full((EPAD - N_EDGES,), NPAD - 1, jnp.int32)
    srcp = jnp.concatenate([src, fill])
    dstp = jnp.concatenate([dst, fill])
    xp = jnp.pad(x[:, 0], (0, NPAD - N_NODES))

    degs = _k_degrees(srcp, dstp)
    p, q, r = _k_norms(xp, degs)
    sv = _k_edges(srcp, dstp, p.reshape(NPAD), q.reshape(NPAD))
    out = _k_reduce(sv, q, r, W1.reshape(WIDTH), b1, W2.reshape(WIDTH), b2)
    return out[0, 0]


# speedup vs baseline: 1.1558x; 1.0004x over previous
import functools

import jax
import jax.numpy as jnp
from jax import lax
from jax.experimental import pallas as pl
from jax.experimental.pallas import tpu as pltpu
from jax.experimental.pallas import tpu_sc as plsc

N_NODES = 100000
N_EDGES = 1600000
WIDTH = 32

NC = 2
NS = 16
NW = NC * NS

CHUNK = 128
SB = 8
BLKE = SB * CHUNK
NPW = 3136
NPAD = NW * NPW
NZS = NPAD // NS
EPW = 50176
EPAD = NW * EPW
NBLOCKS = EPW // BLKE

_mesh = lambda: plsc.VectorSubcoreMesh(core_axis_name="c", subcore_axis_name="s")


def _wid():
    return lax.axis_index("c") * NS + lax.axis_index("s")


def _zero_spmem(zbuf, accs):
    sid = lax.axis_index("s")

    def zb(i, _):
        zbuf[pl.ds(i * 16, 16)] = jnp.zeros((16,), jnp.float32)
        return 0

    lax.fori_loop(0, NZS // 16, zb, 0)
    for acc in accs:
        pltpu.sync_copy(zbuf, acc.at[pl.ds(sid * NZS, NZS)])
    plsc.subcore_barrier()


def _dump_spmem(out_hbm, kind, acc):
    cid = lax.axis_index("c")
    sid = lax.axis_index("s")
    base = (kind * NC + cid) * NPAD + sid * NZS
    pltpu.sync_copy(acc.at[pl.ds(sid * NZS, NZS)], out_hbm.at[pl.ds(base, NZS)])


@functools.partial(
    pl.kernel,
    out_type=jax.ShapeDtypeStruct((2 * NC * NPAD,), jnp.float32),
    mesh=_mesh(),
    scratch_types=[
        [pltpu.VMEM((CHUNK,), jnp.int32)] * SB,
        [pltpu.VMEM((CHUNK,), jnp.int32)] * SB,
        pltpu.VMEM((CHUNK,), jnp.float32),
        pltpu.VMEM((NZS,), jnp.float32),
        pltpu.VMEM_SHARED((NPAD,), jnp.float32),
        pltpu.VMEM_SHARED((NPAD,), jnp.float32),
        pltpu.SemaphoreType.DMA,
        pltpu.SemaphoreType.DMA,
    ],
)
def _k_degrees(src_hbm, dst_hbm, out_hbm, sidx, didx, ones_v, zbuf, acc_o, acc_i,
               sem_i, sem_s):
    def fill(i, _):
        ones_v[pl.ds(i * 16, 16)] = jnp.ones((16,), jnp.float32)
        return 0

    lax.fori_loop(0, CHUNK // 16, fill, 0)
    _zero_spmem(zbuf, (acc_o, acc_i))

    eb = _wid() * EPW

    def body(t, _):
        o = eb + t * BLKE
        ld = []
        for j in range(SB):
            ld.append(pltpu.async_copy(
                src_hbm.at[pl.ds(o + j * CHUNK, CHUNK)], sidx[j], sem_i))
            ld.append(pltpu.async_copy(
                dst_hbm.at[pl.ds(o + j * CHUNK, CHUNK)], didx[j], sem_i))
        for d in ld:
            d.wait()
        sc = []
        for j in range(SB):
            sc.append(pltpu.async_copy(ones_v, acc_o.at[sidx[j]], sem_s, add=True))
            sc.append(pltpu.async_copy(ones_v, acc_i.at[didx[j]], sem_s, add=True))
        for d in sc:
            d.wait()
        return 0

    lax.fori_loop(0, NBLOCKS, body, 0)
    plsc.subcore_barrier()
    _dump_spmem(out_hbm, 0, acc_o)
    _dump_spmem(out_hbm, 1, acc_i)


NROW = NPAD // 128


def _refined_rsqrt(d):
    y = jax.lax.rsqrt(d)
    return y * (1.5 - 0.5 * d * y * y)


def _k_norms_body(xp_ref, degs_ref, p_ref, q_ref, r_ref):
    do = degs_ref[0] + degs_ref[1]
    di = degs_ref[2] + degs_ref[3]
    r = _refined_rsqrt(jnp.where(do > 0.0, do, 1.0))
    q_ref[...] = _refined_rsqrt(jnp.where(di > 0.0, di, 1.0))
    r_ref[...] = r
    p_ref[...] = r * xp_ref[...]


def _k_norms(xp, degs):
    return pl.pallas_call(
        _k_norms_body,
        out_shape=(
            jax.ShapeDtypeStruct((NROW, 128), jnp.float32),
            jax.ShapeDtypeStruct((NROW, 128), jnp.float32),
            jax.ShapeDtypeStruct((NROW, 128), jnp.float32),
        ),
    )(xp.reshape(NROW, 128), degs.reshape(4, NROW, 128))


@functools.partial(
    pl.kernel,
    out_type=jax.ShapeDtypeStruct((2 * NC * NPAD,), jnp.float32),
    mesh=_mesh(),
    scratch_types=[
        [pltpu.VMEM((CHUNK,), jnp.int32)] * SB,
        [pltpu.VMEM((CHUNK,), jnp.int32)] * SB,
        [pltpu.VMEM((CHUNK,), jnp.float32)] * SB,
        [pltpu.VMEM((CHUNK,), jnp.float32)] * SB,
        pltpu.VMEM((NZS,), jnp.float32),
        pltpu.VMEM_SHARED((NPAD,), jnp.float32),
        pltpu.VMEM_SHARED((NPAD,), jnp.float32),
        pltpu.VMEM_SHARED((NPAD,), jnp.float32),
        pltpu.VMEM_SHARED((NPAD,), jnp.float32),
        pltpu.SemaphoreType.DMA,
        pltpu.SemaphoreType.DMA,
        pltpu.SemaphoreType.DMA,
    ],
)
def _k_edges(src_hbm, dst_hbm, p_hbm, q_hbm, out_hbm,
             sidx, didx, pv, qv, zbuf, acc_s, acc_v, p_sp, q_sp, sem_i, sem_g, sem_s):
    sid = lax.axis_index("s")
    sl16 = pl.ds(sid * NZS, NZS)
    pltpu.sync_copy(p_hbm.at[sl16], zbuf)
    pltpu.sync_copy(zbuf, p_sp.at[sl16])
    pltpu.sync_copy(q_hbm.at[sl16], zbuf)
    pltpu.sync_copy(zbuf, q_sp.at[sl16])
    _zero_spmem(zbuf, (acc_s, acc_v))

    eb = _wid() * EPW

    def body(t, _):
        o = eb + t * BLKE
        ld = []
        for j in range(SB):
            ld.append(pltpu.async_copy(
                src_hbm.at[pl.ds(o + j * CHUNK, CHUNK)], sidx[j], sem_i))
            ld.append(pltpu.async_copy(
                dst_hbm.at[pl.ds(o + j * CHUNK, CHUNK)], didx[j], sem_i))
        for d in ld:
            d.wait()
        gp = [pltpu.async_copy(p_sp.at[sidx[j]], pv[j], sem_g) for j in range(SB)]
        for d in gp:
            d.wait()
        sp = [pltpu.async_copy(pv[j], acc_s.at[didx[j]], sem_s, add=True)
              for j in range(SB)]
        gq = [pltpu.async_copy(q_sp.at[didx[j]], qv[j], sem_g) for j in range(SB)]
        for d in gq:
            d.wait()
        sq = [pltpu.async_copy(qv[j], acc_v.at[sidx[j]], sem_s, add=True)
              for j in range(SB)]
        for d in sp + sq:
            d.wait()
        return 0

    lax.fori_loop(0, NBLOCKS, body, 0)
    plsc.subcore_barrier()
    _dump_spmem(out_hbm, 0, acc_s)
    _dump_spmem(out_hbm, 1, acc_v)


def _k_reduce_body(sv_ref, q_ref, r_ref, w1_ref, b1_ref, w2_ref, b2_ref, out_ref):
    s = q_ref[...] * (sv_ref[0] + sv_ref[1])
    v = r_ref[...] * (sv_ref[2] + sv_ref[3])
    g = (jax.lax.broadcasted_iota(jnp.int32, (NROW, 128), 0) * 128
         + jax.lax.broadcasted_iota(jnp.int32, (NROW, 128), 1))
    v = jnp.where(g < N_NODES, v, 0.0)
    t = jnp.zeros((NROW, 128), jnp.float32)
    for j in range(WIDTH):
        t = t + jnp.maximum(s * w1_ref[j] + b1_ref[j], 0.0) * w2_ref[j]
    out_ref[0, 0] = jnp.sum(v * t) / N_NODES + b2_ref[0]


def _k_reduce(sv, q, r, w1, b1, w2, b2):
    return pl.pallas_call(
        _k_reduce_body,
        in_specs=[
            pl.BlockSpec(memory_space=pltpu.MemorySpace.VMEM),
            pl.BlockSpec(memory_space=pltpu.MemorySpace.VMEM),
            pl.BlockSpec(memory_space=pltpu.MemorySpace.VMEM),
            pl.BlockSpec(memory_space=pltpu.MemorySpace.SMEM),
            pl.BlockSpec(memory_space=pltpu.MemorySpace.SMEM),
            pl.BlockSpec(memory_space=pltpu.MemorySpace.SMEM),
            pl.BlockSpec(memory_space=pltpu.MemorySpace.SMEM),
        ],
        out_specs=pl.BlockSpec(memory_space=pltpu.MemorySpace.SMEM),
        out_shape=jax.ShapeDtypeStruct((1, 1), jnp.float32),
    )(sv.reshape(4, NROW, 128), q, r, w1, b1, w2, b2)


def kernel(x, edge_index, W1, b1, W2, b2):
    src = edge_index[0].astype(jnp.int32)
    dst = edge_index[1].astype(jnp.int32)
    fill = jnp.full((EPAD - N_EDGES,), NPAD - 1, jnp.int32)
    srcp = jnp.concatenate([src, fill])
    dstp = jnp.concatenate([dst, fill])
    xp = jnp.pad(x[:, 0], (0, NPAD - N_NODES))

    degs = _k_degrees(srcp, dstp)
    p, q, r = _k_norms(xp, degs)
    sv = _k_edges(srcp, dstp, p.reshape(NPAD), q.reshape(NPAD))
    out = _k_reduce(sv, q, r, W1.reshape(WIDTH), b1, W2.reshape(WIDTH), b2)
    return out[0, 0]
